# baseline (speedup 1.0000x reference)
import jax
import jax.numpy as jnp
from jax import lax
from jax.experimental import pallas as pl
from jax.experimental.pallas import tpu as pltpu
from jax.experimental.pallas import tpu_sc as plsc

EMBED = 100
STATE = 200
LANES = 16
NSUB = 16
PAD = NSUB * LANES
USER_SLOT = 8
CHUNKS = (0, 16, 32, 48, 64, 80, 84)


def _sc_body(gidx_hbm, w_hbm, bias_hbm, utabT_hbm, gtabT_hbm, out_hbm,
             idx_v, w_v, bias_v, col_a, col_b, col_c, col_d, ucol_v, acc_v,
             tot_v, out_v, shared, sem_a, sem_b, sem_c, sem_d, sem_u):
    cid = lax.axis_index("c")
    sid = lax.axis_index("s")
    iota = lax.iota(jnp.int32, LANES)

    base = sid * LANES
    pltpu.sync_copy(gidx_hbm.at[pl.ds(base, LANES)], idx_v)
    pltpu.sync_copy(w_hbm.at[pl.ds(base, LANES)], w_v)

    idx = idx_v[...]
    wvec = w_v[...]

    def _lane(vec, r, zero):
        return jnp.sum(jnp.where(iota == r, vec, zero))

    blks = [pl.multiple_of(_lane(lax.bitwise_and(idx, jnp.int32(~127)), r, 0),
                           128) for r in range(NSUB)]
    lanes = [_lane(lax.bitwise_and(idx, 127), r, 0) for r in range(NSUB)]
    ws = [_lane(wvec, r, 0.0) for r in range(NSUB)]

    @pl.when((cid == 0) & (sid == NSUB - 1))
    def _():
        pltpu.async_copy(utabT_hbm.at[:, pl.ds(blks[USER_SLOT], 128)],
                         ucol_v, sem_u)

    bufs = (col_a, col_b, col_c, col_d)
    sems = (sem_a, sem_b, sem_c, sem_d)
    nbuf = len(bufs)

    def _start(r):
        return pltpu.async_copy(
            gtabT_hbm.at[:, pl.ds(blks[r], 128)], bufs[r % nbuf],
            sems[r % nbuf])

    evecs = [o + iota for o in CHUNKS]
    accs = [jnp.zeros((LANES,), jnp.float32) for _ in CHUNKS]
    inflight = [_start(r) for r in range(nbuf - 1)]
    for r in range(NSUB):
        if r + nbuf - 1 < NSUB:
            inflight.append(_start(r + nbuf - 1))
        inflight.pop(0).wait()
        lvec = jnp.full((LANES,), lanes[r], jnp.int32)
        buf = bufs[r % nbuf]
        for ci in range(len(CHUNKS)):
            val = plsc.load_gather(buf, [evecs[ci], lvec])
            accs[ci] = accs[ci] + val * ws[r]
    for ci, o in enumerate(CHUNKS):
        acc_v[0, pl.ds(o, LANES)] = accs[ci]

    pltpu.sync_copy(acc_v, shared.at[pl.ds(sid, 1)])
    plsc.subcore_barrier()

    @pl.when((cid == 0) & (sid == NSUB - 1))
    def _():
        pltpu.sync_copy(shared, tot_v)
        pltpu.sync_copy(bias_hbm, bias_v)
        pltpu.make_async_copy(utabT_hbm.at[:, pl.ds(blks[USER_SLOT], 128)],
                              ucol_v, sem_u).wait()
        bias = bias_v[...]
        ulvec = jnp.full((LANES,), lanes[USER_SLOT], jnp.int32)
        for ci, o in enumerate(CHUNKS):
            tot = jnp.zeros((LANES,), jnp.float32)
            for r in range(NSUB):
                tot = tot + tot_v[r, pl.ds(o, LANES)]
            wav = tot * (1.0 / EMBED) + bias
            uev = plsc.load_gather(ucol_v, [evecs[ci], ulvec])
            out_v[0, pl.ds(o, LANES)] = uev
            out_v[1, pl.ds(o, LANES)] = uev * wav
            out_v[2, pl.ds(o, LANES)] = wav
        pltpu.sync_copy(out_v, out_hbm)


@jax.jit
def _sc_call(gidx, w, bias16, user_table, game_table):
    utabT = user_table.T
    gtabT = game_table.T
    mesh = plsc.VectorSubcoreMesh(core_axis_name="c", subcore_axis_name="s",
                                  num_cores=2, num_subcores=NSUB)
    out = pl.kernel(
        _sc_body,
        out_type=jax.ShapeDtypeStruct((3, 128), jnp.float32),
        mesh=mesh,
        compiler_params=pltpu.CompilerParams(needs_layout_passes=False),
        scratch_types=[
            pltpu.VMEM((LANES,), jnp.int32),
            pltpu.VMEM((LANES,), jnp.float32),
            pltpu.VMEM((LANES,), jnp.float32),
            pltpu.VMEM((EMBED, 128), jnp.float32),
            pltpu.VMEM((EMBED, 128), jnp.float32),
            pltpu.VMEM((EMBED, 128), jnp.float32),
            pltpu.VMEM((EMBED, 128), jnp.float32),
            pltpu.VMEM((EMBED, 128), jnp.float32),
            pltpu.VMEM((1, 128), jnp.float32),
            pltpu.VMEM((NSUB, 128), jnp.float32),
            pltpu.VMEM((3, 128), jnp.float32),
            pltpu.VMEM_SHARED((NSUB, 128), jnp.float32),
            pltpu.SemaphoreType.DMA,
            pltpu.SemaphoreType.DMA,
            pltpu.SemaphoreType.DMA,
            pltpu.SemaphoreType.DMA,
            pltpu.SemaphoreType.DMA,
        ],
    )(gidx, w, bias16, utabT, gtabT)
    return out


def kernel(user, games, user_table, game_table, wav_w, wav_b):
    gidx = jnp.zeros((PAD,), jnp.int32).at[:STATE].set(games.astype(jnp.int32))
    gidx = gidx.at[(NSUB - 1) * LANES + USER_SLOT].set(user.astype(jnp.int32))
    w = jnp.zeros((PAD,), jnp.float32).at[:STATE].set(
        wav_w.reshape(STATE).astype(jnp.float32))
    bias16 = jnp.broadcast_to(wav_b.astype(jnp.float32), (LANES,))
    out = _sc_call(gidx, w, bias16, user_table, game_table)
    return out[:, :EMBED].reshape(1, 3 * EMBED)


# speedup vs baseline: 1.2320x; 1.2320x over previous
import jax
import jax.numpy as jnp
from jax import lax
from jax.experimental import pallas as pl
from jax.experimental.pallas import tpu as pltpu
from jax.experimental.pallas import tpu_sc as plsc

EMBED = 100
STATE = 200
LANES = 16
NSUB = 16
NPER = 8
HALF = STATE // 2
USER_SLOT = 0
CHUNKS = (0, 16, 32, 48, 64, 80, 84)


def _sc_body(gidx_hbm, w_hbm, bias_hbm, utabT_hbm, gtabT_hbm, out_hbm,
             idx_v, w_v, bias_v, col_a, col_b, col_c, col_d, ucol_v, acc_v,
             tot_v, out_v, shared, sem_a, sem_b, sem_c, sem_d, sem_u):
    cid = lax.axis_index("c")
    sid = lax.axis_index("s")
    iota = lax.iota(jnp.int32, LANES)

    base = (cid * NSUB + sid) * NPER
    base = pl.multiple_of(base, 8)
    pltpu.sync_copy(gidx_hbm.at[pl.ds(base, LANES)], idx_v)
    pltpu.sync_copy(w_hbm.at[pl.ds(base, LANES)], w_v)

    idx = idx_v[...]
    wvec = w_v[...]

    def _lane(vec, r, zero):
        return jnp.sum(jnp.where(iota == r, vec, zero))

    blks = [pl.multiple_of(_lane(lax.bitwise_and(idx, jnp.int32(~127)), r, 0),
                           128) for r in range(NPER)]
    lanes = [_lane(lax.bitwise_and(idx, 127), r, 0) for r in range(NPER)]
    ws = [_lane(wvec, r, 0.0) for r in range(NPER)]

    @pl.when(sid == NSUB - 1)
    def _():
        pltpu.async_copy(utabT_hbm.at[:, pl.ds(blks[USER_SLOT], 128)],
                         ucol_v, sem_u)

    bufs = (col_a, col_b, col_c, col_d)
    sems = (sem_a, sem_b, sem_c, sem_d)
    nbuf = len(bufs)

    def _start(r):
        return pltpu.async_copy(
            gtabT_hbm.at[:, pl.ds(blks[r], 128)], bufs[r % nbuf],
            sems[r % nbuf])

    evecs = [o + iota for o in CHUNKS]
    accs = [jnp.zeros((LANES,), jnp.float32) for _ in CHUNKS]
    inflight = [_start(r) for r in range(min(nbuf - 1, NPER))]
    for r in range(NPER):
        if r + nbuf - 1 < NPER:
            inflight.append(_start(r + nbuf - 1))
        inflight.pop(0).wait()
        lvec = jnp.full((LANES,), lanes[r], jnp.int32)
        buf = bufs[r % nbuf]
        for ci in range(len(CHUNKS)):
            val = plsc.load_gather(buf, [evecs[ci], lvec])
            accs[ci] = accs[ci] + val * ws[r]
    for ci, o in enumerate(CHUNKS):
        acc_v[0, pl.ds(o, LANES)] = accs[ci]

    pltpu.sync_copy(acc_v, shared.at[pl.ds(sid, 1)])
    plsc.subcore_barrier()

    @pl.when(sid == NSUB - 1)
    def _():
        pltpu.sync_copy(shared, tot_v)
        pltpu.sync_copy(bias_hbm, bias_v)
        pltpu.make_async_copy(utabT_hbm.at[:, pl.ds(blks[USER_SLOT], 128)],
                              ucol_v, sem_u).wait()
        bias = jnp.where(cid == 0, 1.0, 0.0) * bias_v[...]
        ulvec = jnp.full((LANES,), lanes[USER_SLOT], jnp.int32)
        for ci, o in enumerate(CHUNKS):
            tot = jnp.zeros((LANES,), jnp.float32)
            for r in range(NSUB):
                tot = tot + tot_v[r, pl.ds(o, LANES)]
            wav = tot * (1.0 / EMBED) + bias
            uev = plsc.load_gather(ucol_v, [evecs[ci], ulvec])
            out_v[0, pl.ds(o, LANES)] = uev
            out_v[1, pl.ds(o, LANES)] = uev * wav
            out_v[2, pl.ds(o, LANES)] = wav
        pltpu.sync_copy(out_v, out_hbm.at[cid])


@jax.jit
def _sc_call(gidx, w, bias16, user_table, game_table):
    utabT = user_table.T
    gtabT = game_table.T
    mesh = plsc.VectorSubcoreMesh(core_axis_name="c", subcore_axis_name="s",
                                  num_cores=2, num_subcores=NSUB)
    out = pl.kernel(
        _sc_body,
        out_type=jax.ShapeDtypeStruct((2, 3, 128), jnp.float32),
        mesh=mesh,
        compiler_params=pltpu.CompilerParams(needs_layout_passes=False),
        scratch_types=[
            pltpu.VMEM((LANES,), jnp.int32),
            pltpu.VMEM((LANES,), jnp.float32),
            pltpu.VMEM((LANES,), jnp.float32),
            pltpu.VMEM((EMBED, 128), jnp.float32),
            pltpu.VMEM((EMBED, 128), jnp.float32),
            pltpu.VMEM((EMBED, 128), jnp.float32),
            pltpu.VMEM((EMBED, 128), jnp.float32),
            pltpu.VMEM((EMBED, 128), jnp.float32),
            pltpu.VMEM((1, 128), jnp.float32),
            pltpu.VMEM((NSUB, 128), jnp.float32),
            pltpu.VMEM((3, 128), jnp.float32),
            pltpu.VMEM_SHARED((NSUB, 128), jnp.float32),
            pltpu.SemaphoreType.DMA,
            pltpu.SemaphoreType.DMA,
            pltpu.SemaphoreType.DMA,
            pltpu.SemaphoreType.DMA,
            pltpu.SemaphoreType.DMA,
        ],
    )(gidx, w, bias16, utabT, gtabT)
    return out


def kernel(user, games, user_table, game_table, wav_w, wav_b):
    games = games.astype(jnp.int32)
    user = user.astype(jnp.int32)
    wflat = wav_w.reshape(STATE).astype(jnp.float32)
    gidx = jnp.zeros((2 * NSUB * NPER + LANES,), jnp.int32)
    gidx = gidx.at[:HALF].set(games[:HALF])
    gidx = gidx.at[128:128 + HALF].set(games[HALF:])
    gidx = gidx.at[120].set(user).at[248].set(user)
    w = jnp.zeros((2 * NSUB * NPER + LANES,), jnp.float32)
    w = w.at[:HALF].set(wflat[:HALF])
    w = w.at[128:128 + HALF].set(wflat[HALF:])
    bias16 = jnp.broadcast_to(wav_b.astype(jnp.float32), (LANES,))
    out = _sc_call(gidx, w, bias16, user_table, game_table)
    combined = jnp.concatenate(
        [out[0, :1], out[0, 1:] + out[1, 1:]], axis=0)
    return combined[:, :EMBED].reshape(1, 3 * EMBED)
